# Initial kernel scaffold; baseline (speedup 1.0000x reference)
import jax
import jax.numpy as jnp
from jax.experimental import pallas as pl


def kernel(logits, norms, labels):
    raise NotImplementedError("write your pallas kernel here")


# speedup vs baseline: 1.1667x; 1.1667x over previous
import math

import jax
import jax.numpy as jnp
from jax.experimental import pallas as pl

B = 1024
C = 100000
C_BLK = 2048

_M = 0.4
_H = 0.333
_S = 64.0
_EPS = 0.001


def _adaact_block(x_ref, n_ref, l_ref, o_ref):
    j = pl.program_id(0)
    col0 = j * C_BLK

    x = x_ref[...]
    xc = jnp.clip(x, -1.0 + _EPS, 1.0 - _EPS)

    sn = jnp.clip(n_ref[...], 0.001, 100.0)
    mean_z = jnp.mean(sn)
    std_z = jnp.sqrt(jnp.sum((sn - mean_z) ** 2) / (B - 1))
    z = jnp.clip((sn - mean_z) / (std_z + _EPS) * _H, -1.0, 1.0)

    lab = l_ref[...]
    cols = col0 + jax.lax.broadcasted_iota(jnp.int32, (B, C_BLK), 1)
    mask = cols == lab

    t = jnp.sum(jnp.where(mask, xc, 0.0), axis=1, keepdims=True)

    g = -_M * z
    cg = jnp.cos(g)
    sg = jnp.sin(g)
    cos_sum = t * cg - jnp.sqrt(jnp.maximum(1.0 - t * t, 0.0)) * sg
    lo_x = _EPS - g
    cond_lo = (lo_x > 0) & (t > jnp.cos(lo_x))
    hi_c = -jnp.cos(_EPS + g)
    cond_hi = (_EPS + g > 0) & (t < hi_c)
    t_ang = jnp.where(cond_lo, jnp.cos(lo_x), jnp.where(cond_hi, hi_c, cos_sum))
    t_add = t_ang - (_M + _M * z)
    gap = 1.0 - _M * z - _M - jnp.cos(_M * z)
    cond_pos = (g >= 0) | (t < cg)
    finalv = jnp.where(cond_pos, t_add, t + gap)

    o_ref[...] = jnp.where(mask, finalv * _S, xc * _S)


def kernel(logits, norms, labels):
    norms2d = norms.reshape(B, 1)
    labels2d = labels.reshape(B, 1)
    grid = (pl.cdiv(C, C_BLK),)
    return pl.pallas_call(
        _adaact_block,
        grid=grid,
        in_specs=[
            pl.BlockSpec((B, C_BLK), lambda j: (0, j)),
            pl.BlockSpec((B, 1), lambda j: (0, 0)),
            pl.BlockSpec((B, 1), lambda j: (0, 0)),
        ],
        out_specs=pl.BlockSpec((B, C_BLK), lambda j: (0, j)),
        out_shape=jax.ShapeDtypeStruct((B, C), jnp.float32),
    )(logits, norms2d, labels2d)


# speedup vs baseline: 1.1993x; 1.0280x over previous
import math

import jax
import jax.numpy as jnp
from jax.experimental import pallas as pl

B = 1024
C = 100000
C_BLK = 2048

_M = 0.4
_H = 0.333
_S = 64.0
_EPS = 0.001


def _adaact_block(x_ref, n_ref, l_ref, o_ref):
    j = pl.program_id(0)
    col0 = j * C_BLK

    x = x_ref[...]
    y = jnp.clip(x * _S, (-1.0 + _EPS) * _S, (1.0 - _EPS) * _S)

    sn = jnp.clip(n_ref[...], 0.001, 100.0)
    mean_z = jnp.mean(sn)
    std_z = jnp.sqrt(jnp.sum((sn - mean_z) ** 2) / (B - 1))
    z = jnp.clip((sn - mean_z) / (std_z + _EPS) * _H, -1.0, 1.0)

    lab = l_ref[...]
    iota = jax.lax.broadcasted_iota(jnp.int32, (B, C_BLK), 1)
    mask = iota == (lab - col0)

    t = jnp.sum(jnp.where(mask, y, 0.0), axis=1, keepdims=True) * (1.0 / _S)

    g = -_M * z
    cg = jnp.cos(g)
    sg = jnp.sin(g)
    cos_sum = t * cg - jnp.sqrt(jnp.maximum(1.0 - t * t, 0.0)) * sg
    lo_x = _EPS - g
    cond_lo = (lo_x > 0) & (t > jnp.cos(lo_x))
    hi_c = -jnp.cos(_EPS + g)
    cond_hi = (_EPS + g > 0) & (t < hi_c)
    t_ang = jnp.where(cond_lo, jnp.cos(lo_x), jnp.where(cond_hi, hi_c, cos_sum))
    t_add = t_ang - (_M + _M * z)
    gap = 1.0 - _M * z - _M - jnp.cos(_M * z)
    cond_pos = (g >= 0) | (t < cg)
    finalv = jnp.where(cond_pos, t_add, t + gap)

    o_ref[...] = jnp.where(mask, finalv * _S, y)


def kernel(logits, norms, labels):
    norms2d = norms.reshape(B, 1)
    labels2d = labels.reshape(B, 1)
    grid = (pl.cdiv(C, C_BLK),)
    return pl.pallas_call(
        _adaact_block,
        grid=grid,
        in_specs=[
            pl.BlockSpec((B, C_BLK), lambda j: (0, j)),
            pl.BlockSpec((B, 1), lambda j: (0, 0)),
            pl.BlockSpec((B, 1), lambda j: (0, 0)),
        ],
        out_specs=pl.BlockSpec((B, C_BLK), lambda j: (0, j)),
        out_shape=jax.ShapeDtypeStruct((B, C), jnp.float32),
    )(logits, norms2d, labels2d)
